# baseline (speedup 1.0000x reference)
import functools

import jax
import jax.numpy as jnp
from jax import lax
from jax.experimental import pallas as pl
from jax.experimental.pallas import tpu as pltpu
from jax.experimental.pallas import tpu_sc as plsc

_D = 64
_C = 128
_NBUF = 4


@functools.lru_cache(maxsize=None)
def _build(B, S):
    info = plsc.get_sparse_core_info()
    NC, NS = info.num_cores, info.num_subcores
    NW = NC * NS
    NBT = B // _C
    T = S * NBT
    per_w = T // NW
    assert T % NW == 0 and per_w % _NBUF == 0
    mesh = plsc.VectorSubcoreMesh(core_axis_name="c", subcore_axis_name="s")

    @functools.partial(
        pl.kernel,
        mesh=mesh,
        out_type=jax.ShapeDtypeStruct((S, _D // 8, NBT, 8, _C), jnp.float32),
        scratch_types=[
            pltpu.VMEM((_NBUF, _C), jnp.int32),
            pltpu.VMEM((_NBUF, _C, _D), jnp.float32),
            pltpu.VMEM((2, _D // 8, 8, _C), jnp.float32),
        ]
        + [pltpu.SemaphoreType.DMA] * (_NBUF + 2),
        compiler_params=pltpu.CompilerParams(
            use_tc_tiling_on_sc=False, needs_layout_passes=False
        ),
    )
    def grab(idx_hbm, table_hbm, out_hbm, idx_v, rows_v, tout_v, *sems):
        gsems, wsems = sems[:_NBUF], sems[_NBUF:]
        wid = lax.axis_index("s") * NC + lax.axis_index("c")
        t0 = wid * per_w

        bvecs = [lax.iota(jnp.int32, 16) + 16 * c for c in range(8)]

        def stage_and_fire(t, slot):
            pltpu.sync_copy(idx_hbm.at[pl.ds(t * _C, _C)], idx_v.at[slot])
            pltpu.async_copy(
                table_hbm.at[idx_v.at[slot]], rows_v.at[slot], gsems[slot]
            )

        def out_slice(t):
            s = t // NBT
            bt = lax.rem(t, NBT)
            return out_hbm.at[s, :, bt]

        for k in range(_NBUF):
            stage_and_fire(t0 + k, k)

        @pl.loop(0, per_w, step=_NBUF)
        def _(g0):
            for b in range(_NBUF):
                g = g0 + b
                t = t0 + g
                ws = b % 2
                pltpu.make_async_copy(
                    table_hbm.at[idx_v.at[b]], rows_v.at[b], gsems[b]
                ).wait()

                @pl.when(g >= 2)
                def _():
                    pltpu.make_async_copy(
                        tout_v.at[ws], out_slice(t), wsems[ws]
                    ).wait()

                rows2d = rows_v.at[b]
                for dt in range(_D // 8):
                    for di in range(8):
                        dvec = jnp.full((16,), dt * 8 + di, jnp.int32)
                        for c in range(8):
                            v = plsc.load_gather(rows2d, [bvecs[c], dvec])
                            tout_v[ws, dt, di, pl.ds(16 * c, 16)] = v

                pltpu.async_copy(tout_v.at[ws], out_slice(t), wsems[ws])

                nf = g + _NBUF

                @pl.when(nf < per_w)
                def _():
                    stage_and_fire(t0 + nf, b)

        for ws in range(2):
            pltpu.make_async_copy(
                tout_v.at[ws], out_hbm.at[0, :, 0], wsems[ws]
            ).wait()

    return grab


def kernel(token_ids, weight):
    B, S = token_ids.shape
    idx_flat = token_ids.T.reshape(-1).astype(jnp.int32)
    out5 = _build(B, S)(idx_flat, weight)
    return out5.transpose(2, 4, 0, 1, 3).reshape(B, S, _D)


# speedup vs baseline: 1.5916x; 1.5916x over previous
import functools

import jax
import jax.numpy as jnp
from jax import lax
from jax.experimental import pallas as pl
from jax.experimental.pallas import tpu as pltpu
from jax.experimental.pallas import tpu_sc as plsc

_D = 64
_C = 128
_NBUF = 4


@functools.lru_cache(maxsize=None)
def _build(B, S):
    info = plsc.get_sparse_core_info()
    NC, NS = info.num_cores, info.num_subcores
    NW = NC * NS
    NBT = B // _C
    T = S * NBT
    per_w = T // NW
    assert T % NW == 0 and per_w % _NBUF == 0
    mesh = plsc.VectorSubcoreMesh(core_axis_name="c", subcore_axis_name="s")

    @functools.partial(
        pl.kernel,
        mesh=mesh,
        out_type=jax.ShapeDtypeStruct((S, _D // 8, NBT, 8, _C), jnp.float32),
        scratch_types=[
            pltpu.VMEM((_NBUF, _C), jnp.int32),
            pltpu.VMEM((_NBUF, _C, _D), jnp.float32),
            pltpu.VMEM((2, _D // 8, 8, _C), jnp.float32),
        ]
        + [pltpu.SemaphoreType.DMA] * (_NBUF + 2),
        compiler_params=pltpu.CompilerParams(
            use_tc_tiling_on_sc=False,
            needs_layout_passes=False,
            disable_bounds_checks=True,
        ),
    )
    def grab(idx_hbm, table_hbm, out_hbm, idx_v, rows_v, tout_v, *sems):
        gsems, wsems = sems[:_NBUF], sems[_NBUF:]
        wid = lax.axis_index("s") * NC + lax.axis_index("c")
        t0 = wid * per_w

        bvecs = [lax.iota(jnp.int32, 16) + 16 * c for c in range(8)]

        def stage_and_fire(t, slot):
            pltpu.sync_copy(idx_hbm.at[pl.ds(t * _C, _C)], idx_v.at[slot])
            pltpu.async_copy(
                table_hbm.at[idx_v.at[slot]], rows_v.at[slot], gsems[slot]
            )

        def out_slice(t):
            s = t // NBT
            bt = lax.rem(t, NBT)
            return out_hbm.at[s, :, bt]

        for k in range(_NBUF):
            stage_and_fire(t0 + k, k)

        @pl.loop(0, per_w, step=_NBUF)
        def _(g0):
            for b in range(_NBUF):
                g = g0 + b
                t = t0 + g
                ws = b % 2
                pltpu.make_async_copy(
                    table_hbm.at[idx_v.at[b]], rows_v.at[b], gsems[b]
                ).wait()

                @pl.when(g >= 2)
                def _():
                    pltpu.make_async_copy(
                        tout_v.at[ws], out_slice(t), wsems[ws]
                    ).wait()

                rows2d = rows_v.at[b]

                @plsc.parallel_loop(0, _D, unroll=8)
                def _(d):
                    dt = d // 8
                    di = lax.rem(d, 8)
                    dvec = jnp.full((16,), d, jnp.int32)
                    for c in range(8):
                        v = plsc.load_gather(rows2d, [bvecs[c], dvec])
                        tout_v[ws, dt, di, pl.ds(16 * c, 16)] = v

                pltpu.async_copy(tout_v.at[ws], out_slice(t), wsems[ws])

                nf = g + _NBUF

                @pl.when(nf < per_w)
                def _():
                    stage_and_fire(t0 + nf, b)

        for ws in range(2):
            pltpu.make_async_copy(
                tout_v.at[ws], out_hbm.at[0, :, 0], wsems[ws]
            ).wait()

    return grab


def kernel(token_ids, weight):
    B, S = token_ids.shape
    idx_flat = token_ids.T.reshape(-1).astype(jnp.int32)
    out5 = _build(B, S)(idx_flat, weight)
    return out5.transpose(2, 4, 0, 1, 3).reshape(B, S, _D)
